# baseline (speedup 1.0000x reference)
import jax
import jax.numpy as jnp
from jax.experimental import pallas as pl
from jax.experimental.pallas import tpu as pltpu

_B = 8
_NC = 100
_NCTX = 10
_D = 512
_SUF = 66
_TKN = 77
_CT = 20
_NSTEP = _NC // _CT


def _body(im_ref, ctx_ref, pre_ref, suf_hbm, w1_ref, b1_ref, w2_ref, b2_ref,
          out_ref, ctxp_ref, suf_bufs, sems):
    c = pl.program_id(0)

    @pl.when(c == 0)
    def _():
        pltpu.make_async_copy(
            suf_hbm.at[pl.ds(0, _CT)], suf_bufs.at[0], sems.at[0]).start()
        h = jnp.maximum(
            jnp.dot(im_ref[:], w1_ref[:], preferred_element_type=jnp.float32)
            + b1_ref[:], 0.0)
        bias = jnp.dot(h, w2_ref[:], preferred_element_type=jnp.float32) + b2_ref[:]
        ctxp_ref[:] = (ctx_ref[:][:, None, :] + bias[None, :, :]).reshape(
            _NCTX * _B, _D)

    @pl.when(c + 1 < _NSTEP)
    def _():
        pltpu.make_async_copy(
            suf_hbm.at[pl.ds((c + 1) * _CT, _CT)],
            suf_bufs.at[(c + 1) % 2], sems.at[(c + 1) % 2]).start()

    out_ref[:, 0:_B, :] = jnp.broadcast_to(
        pre_ref[:].reshape(_CT, 1, _D), (_CT, _B, _D))
    out_ref[:, _B:_B * (1 + _NCTX), :] = jnp.broadcast_to(
        ctxp_ref[:][None], (_CT, _NCTX * _B, _D))

    pltpu.make_async_copy(
        suf_hbm.at[pl.ds(c * _CT, _CT)], suf_bufs.at[c % 2], sems.at[c % 2]
    ).wait()
    suf = suf_bufs[c % 2]
    out_ref[:, _B * (1 + _NCTX):, :] = jnp.broadcast_to(
        suf[:, :, None, :], (_CT, _SUF, _B, _D)).reshape(_CT, _SUF * _B, _D)


def kernel(im_features, ctx, token_prefix, token_suffix, W1, b1, W2, b2):
    out_p = pl.pallas_call(
        _body,
        grid=(_NSTEP,),
        in_specs=[
            pl.BlockSpec((_B, _D), lambda c: (0, 0)),
            pl.BlockSpec((_NCTX, _D), lambda c: (0, 0)),
            pl.BlockSpec((_CT, 1, _D), lambda c: (c, 0, 0)),
            pl.BlockSpec(memory_space=pltpu.HBM),
            pl.BlockSpec((_D, _D // 4), lambda c: (0, 0)),
            pl.BlockSpec((1, _D // 4), lambda c: (0, 0)),
            pl.BlockSpec((_D // 4, _D), lambda c: (0, 0)),
            pl.BlockSpec((1, _D), lambda c: (0, 0)),
        ],
        out_specs=pl.BlockSpec((_CT, _TKN * _B, _D), lambda c: (c, 0, 0)),
        out_shape=jax.ShapeDtypeStruct((_NC, _TKN * _B, _D), jnp.float32),
        scratch_shapes=[
            pltpu.VMEM((_NCTX * _B, _D), jnp.float32),
            pltpu.VMEM((2, _CT, _SUF, _D), jnp.float32),
            pltpu.SemaphoreType.DMA((2,)),
        ],
    )(im_features, ctx, token_prefix, token_suffix, W1,
      b1.reshape(1, -1), W2, b2.reshape(1, -1))
    return out_p.reshape(_NC, _TKN, _B, _D).transpose(2, 0, 1, 3)


# speedup vs baseline: 1.0092x; 1.0092x over previous
import jax
import jax.numpy as jnp
from jax.experimental import pallas as pl
from jax.experimental.pallas import tpu as pltpu

_B = 8
_NC = 100
_NCTX = 10
_D = 512
_SUF = 66
_TKN = 77
_CT = 10
_NSTEP = _NC // _CT
_VRES = 12288


def _body(im_ref, ctx_ref, pre_ref, suf_hbm, w1_ref, b1_ref, w2_ref, b2_ref,
          out_ref, ctxp_ref, suf_bufs, sems, vmem_reserve):
    c = pl.program_id(0)

    @pl.when(c == 0)
    def _():
        pltpu.make_async_copy(
            suf_hbm.at[pl.ds(0, _CT)], suf_bufs.at[0], sems.at[0]).start()
        h = jnp.maximum(
            jnp.dot(im_ref[:], w1_ref[:], preferred_element_type=jnp.float32)
            + b1_ref[:], 0.0)
        bias = jnp.dot(h, w2_ref[:], preferred_element_type=jnp.float32) + b2_ref[:]
        ctxp_ref[:] = (ctx_ref[:][:, None, :] + bias[None, :, :]).reshape(
            _NCTX * _B, _D)
        vmem_reserve[0:1, :] = b2_ref[:]

    @pl.when(c + 1 < _NSTEP)
    def _():
        pltpu.make_async_copy(
            suf_hbm.at[pl.ds((c + 1) * _CT, _CT)],
            suf_bufs.at[(c + 1) % 2], sems.at[(c + 1) % 2]).start()

    out_ref[:, 0:_B, :] = jnp.broadcast_to(
        pre_ref[:].reshape(_CT, 1, _D), (_CT, _B, _D))
    out_ref[:, _B:_B * (1 + _NCTX), :] = jnp.broadcast_to(
        ctxp_ref[:][None], (_CT, _NCTX * _B, _D))

    pltpu.make_async_copy(
        suf_hbm.at[pl.ds(c * _CT, _CT)], suf_bufs.at[c % 2], sems.at[c % 2]
    ).wait()
    suf = suf_bufs[c % 2]
    out_ref[:, _B * (1 + _NCTX):, :] = jnp.broadcast_to(
        suf[:, :, None, :], (_CT, _SUF, _B, _D)).reshape(_CT, _SUF * _B, _D)


def kernel(im_features, ctx, token_prefix, token_suffix, W1, b1, W2, b2):
    out_p = pl.pallas_call(
        _body,
        grid=(_NSTEP,),
        in_specs=[
            pl.BlockSpec((_B, _D), lambda c: (0, 0)),
            pl.BlockSpec((_NCTX, _D), lambda c: (0, 0)),
            pl.BlockSpec((_CT, 1, _D), lambda c: (c, 0, 0)),
            pl.BlockSpec(memory_space=pltpu.HBM),
            pl.BlockSpec((_D, _D // 4), lambda c: (0, 0)),
            pl.BlockSpec((1, _D // 4), lambda c: (0, 0)),
            pl.BlockSpec((_D // 4, _D), lambda c: (0, 0)),
            pl.BlockSpec((1, _D), lambda c: (0, 0)),
        ],
        out_specs=pl.BlockSpec((_CT, _TKN * _B, _D), lambda c: (c, 0, 0)),
        out_shape=jax.ShapeDtypeStruct((_NC, _TKN * _B, _D), jnp.float32),
        scratch_shapes=[
            pltpu.VMEM((_NCTX * _B, _D), jnp.float32),
            pltpu.VMEM((2, _CT, _SUF, _D), jnp.float32),
            pltpu.SemaphoreType.DMA((2,)),
            pltpu.VMEM((_VRES, _D), jnp.float32),
        ],
    )(im_features, ctx, token_prefix, token_suffix, W1,
      b1.reshape(1, -1), W2, b2.reshape(1, -1))
    return out_p.reshape(_NC, _TKN, _B, _D).transpose(2, 0, 1, 3)


# speedup vs baseline: 1.0237x; 1.0145x over previous
import jax
import jax.numpy as jnp
from jax.experimental import pallas as pl
from jax.experimental.pallas import tpu as pltpu

_B = 8
_NC = 100
_NCTX = 10
_D = 512
_SUF = 66
_TKN = 77
_CT = 10
_NSTEP = _NC // _CT
_DEPTH = 4
_CTXROWS = 9216


def _body(im_ref, ctx_ref, pre_ref, suf_hbm, w1_ref, b1_ref, w2_ref, b2_ref,
          out_ref, ctxp_ref, suf_bufs, sems):
    c = pl.program_id(0)

    @pl.when(c == 0)
    def _():
        for k in range(_DEPTH - 1):
            pltpu.make_async_copy(
                suf_hbm.at[pl.ds(k * _CT, _CT)], suf_bufs.at[k],
                sems.at[k]).start()
        h = jnp.maximum(
            jnp.dot(im_ref[:], w1_ref[:], preferred_element_type=jnp.float32)
            + b1_ref[:], 0.0)
        bias = jnp.dot(h, w2_ref[:], preferred_element_type=jnp.float32) + b2_ref[:]
        ctxp_ref[0:_NCTX * _B, :] = (
            ctx_ref[:][:, None, :] + bias[None, :, :]).reshape(_NCTX * _B, _D)

    @pl.when(c + _DEPTH - 1 < _NSTEP)
    def _():
        pltpu.make_async_copy(
            suf_hbm.at[pl.ds((c + _DEPTH - 1) * _CT, _CT)],
            suf_bufs.at[(c + _DEPTH - 1) % _DEPTH],
            sems.at[(c + _DEPTH - 1) % _DEPTH]).start()

    out_ref[:, 0:_B, :] = jnp.broadcast_to(
        pre_ref[:].reshape(_CT, 1, _D), (_CT, _B, _D))
    out_ref[:, _B:_B * (1 + _NCTX), :] = jnp.broadcast_to(
        ctxp_ref[0:_NCTX * _B, :][None], (_CT, _NCTX * _B, _D))

    pltpu.make_async_copy(
        suf_hbm.at[pl.ds(c * _CT, _CT)], suf_bufs.at[c % _DEPTH],
        sems.at[c % _DEPTH]).wait()
    suf = suf_bufs[c % _DEPTH]
    out_ref[:, _B * (1 + _NCTX):, :] = jnp.broadcast_to(
        suf[:, :, None, :], (_CT, _SUF, _B, _D)).reshape(_CT, _SUF * _B, _D)


def kernel(im_features, ctx, token_prefix, token_suffix, W1, b1, W2, b2):
    out_p = pl.pallas_call(
        _body,
        grid=(_NSTEP,),
        in_specs=[
            pl.BlockSpec((_B, _D), lambda c: (0, 0)),
            pl.BlockSpec((_NCTX, _D), lambda c: (0, 0)),
            pl.BlockSpec((_CT, 1, _D), lambda c: (c, 0, 0)),
            pl.BlockSpec(memory_space=pltpu.HBM),
            pl.BlockSpec((_D, _D // 4), lambda c: (0, 0)),
            pl.BlockSpec((1, _D // 4), lambda c: (0, 0)),
            pl.BlockSpec((_D // 4, _D), lambda c: (0, 0)),
            pl.BlockSpec((1, _D), lambda c: (0, 0)),
        ],
        out_specs=pl.BlockSpec((_CT, _TKN * _B, _D), lambda c: (c, 0, 0)),
        out_shape=jax.ShapeDtypeStruct((_NC, _TKN * _B, _D), jnp.float32),
        scratch_shapes=[
            pltpu.VMEM((_CTXROWS, _D), jnp.float32),
            pltpu.VMEM((_DEPTH, _CT, _SUF, _D), jnp.float32),
            pltpu.SemaphoreType.DMA((_DEPTH,)),
        ],
    )(im_features, ctx, token_prefix, token_suffix, W1,
      b1.reshape(1, -1), W2, b2.reshape(1, -1))
    return out_p.reshape(_NC, _TKN, _B, _D).transpose(2, 0, 1, 3)


# speedup vs baseline: 1.0273x; 1.0035x over previous
import jax
import jax.numpy as jnp
from jax.experimental import pallas as pl
from jax.experimental.pallas import tpu as pltpu

_B = 8
_NC = 100
_NCTX = 10
_D = 512
_SUF = 66
_TKN = 77
_CT = 10
_NSTEP = _NC // _CT
_DEPTH = 4
_CTXROWS = 11264


def _body(im_ref, ctx_ref, pre_ref, suf_hbm, w1_ref, b1_ref, w2_ref, b2_ref,
          out_ref, ctxp_ref, suf_bufs, sems):
    c = pl.program_id(0)

    @pl.when(c == 0)
    def _():
        for k in range(_DEPTH - 1):
            pltpu.make_async_copy(
                suf_hbm.at[pl.ds(k * _CT, _CT)], suf_bufs.at[k],
                sems.at[k]).start()
        h = jnp.maximum(
            jnp.dot(im_ref[:], w1_ref[:], preferred_element_type=jnp.float32)
            + b1_ref[:], 0.0)
        bias = jnp.dot(h, w2_ref[:], preferred_element_type=jnp.float32) + b2_ref[:]
        ctxp_ref[0:_NCTX * _B, :] = (
            ctx_ref[:][:, None, :] + bias[None, :, :]).reshape(_NCTX * _B, _D)

    @pl.when(c + _DEPTH - 1 < _NSTEP)
    def _():
        pltpu.make_async_copy(
            suf_hbm.at[pl.ds((c + _DEPTH - 1) * _CT, _CT)],
            suf_bufs.at[(c + _DEPTH - 1) % _DEPTH],
            sems.at[(c + _DEPTH - 1) % _DEPTH]).start()

    out_ref[:, 0:_B, :] = jnp.broadcast_to(
        pre_ref[:].reshape(_CT, 1, _D), (_CT, _B, _D))
    out_ref[:, _B:_B * (1 + _NCTX), :] = jnp.broadcast_to(
        ctxp_ref[0:_NCTX * _B, :][None], (_CT, _NCTX * _B, _D))

    pltpu.make_async_copy(
        suf_hbm.at[pl.ds(c * _CT, _CT)], suf_bufs.at[c % _DEPTH],
        sems.at[c % _DEPTH]).wait()
    suf = suf_bufs[c % _DEPTH]
    out_ref[:, _B * (1 + _NCTX):, :] = jnp.broadcast_to(
        suf[:, :, None, :], (_CT, _SUF, _B, _D)).reshape(_CT, _SUF * _B, _D)


def kernel(im_features, ctx, token_prefix, token_suffix, W1, b1, W2, b2):
    out_p = pl.pallas_call(
        _body,
        grid=(_NSTEP,),
        in_specs=[
            pl.BlockSpec((_B, _D), lambda c: (0, 0)),
            pl.BlockSpec((_NCTX, _D), lambda c: (0, 0)),
            pl.BlockSpec((_CT, 1, _D), lambda c: (c, 0, 0)),
            pl.BlockSpec(memory_space=pltpu.HBM),
            pl.BlockSpec((_D, _D // 4), lambda c: (0, 0)),
            pl.BlockSpec((1, _D // 4), lambda c: (0, 0)),
            pl.BlockSpec((_D // 4, _D), lambda c: (0, 0)),
            pl.BlockSpec((1, _D), lambda c: (0, 0)),
        ],
        out_specs=pl.BlockSpec((_CT, _TKN * _B, _D), lambda c: (c, 0, 0)),
        out_shape=jax.ShapeDtypeStruct((_NC, _TKN * _B, _D), jnp.float32),
        scratch_shapes=[
            pltpu.VMEM((_CTXROWS, _D), jnp.float32),
            pltpu.VMEM((_DEPTH, _CT, _SUF, _D), jnp.float32),
            pltpu.SemaphoreType.DMA((_DEPTH,)),
        ],
    )(im_features, ctx, token_prefix, token_suffix, W1,
      b1.reshape(1, -1), W2, b2.reshape(1, -1))
    return out_p.reshape(_NC, _TKN, _B, _D).transpose(2, 0, 1, 3)
